# baseline (speedup 1.0000x reference)
import jax
import jax.numpy as jnp
from jax import lax
from jax.experimental import pallas as pl
from jax.experimental.pallas import tpu as pltpu
from jax.experimental.pallas import tpu_sc as plsc

_N = 10000
_CH = 128
_G = 16
_EPS = 1e-6

_NC = 2
_NS = 16
_NW = _NC * _NS
_CHK = 64
_NCHT = 157
_EPT = _NCHT * _CHK
_EPAD = _NW * _EPT
_NRCH = 157
_RTAIL = _N - (_NRCH - 1) * _CHK
_MPT = 10

_BLK = 2000
_NB = _N // _BLK

_HI = lax.Precision.HIGHEST


def _sig(z):
    return 1.0 / (1.0 + jnp.exp(-z))


def _dot(a, b):
    return lax.dot_general(a, b, (((1,), (0,)), ((), ())), precision=_HI)


def _dotT(a, b):
    return lax.dot_general(a, b, (((0,), (0,)), ((), ())), precision=_HI)


def _onehot(b):
    ids = lax.broadcasted_iota(jnp.int32, (1, _G), 1).astype(jnp.float32)
    return (b == ids).astype(jnp.float32)


def _stats_body(x_ref, b_ref, ms_ref):
    x = x_ref[...]
    oh = _onehot(b_ref[...])
    stats = _dotT(oh, x * x)
    counts = _dotT(oh, jnp.ones_like(b_ref[...]))
    ms_ref[...] = stats / jnp.maximum(counts, 1.0)


def _a2_body(x_ref, b_ref, f_ref, ms_ref, t_ref, wt_ref, bt_ref, wf_ref,
             wm1_ref, bm1_ref, wm2_ref, bm2_ref, wr_ref, wo_ref,
             p_ref, r_ref):
    x = x_ref[...]
    oh = _onehot(b_ref[...])
    inv = lax.rsqrt(_dot(oh, ms_ref[...]) + _EPS)
    h = x * inv
    tv = t_ref[...]
    st = _dot(tv * _sig(tv), wt_ref[...]) + bt_ref[...]
    cond = _dot(oh, st) + f_ref[...] * wf_ref[...]
    gamma = cond[:, :_CH]
    beta = cond[:, _CH:]
    h = h * (1.0 + gamma) + beta
    h = h * _sig(h)
    u = _dot(h, wm1_ref[...]) + bm1_ref[...]
    u = u * _sig(u)
    h2 = h + _dot(u, wm2_ref[...]) + bm2_ref[...]
    p_ref[...] = _dot(h2, wr_ref[...])
    r_ref[...] = _dot(h2, wo_ref[...])


def _b_body(sa_ref, sb_ref, rt_ref, wr_ref, wo_ref, p_ref, r_ref):
    h = sa_ref[...] + sb_ref[...] + rt_ref[...]
    h = h * _sig(h)
    p_ref[...] = _dot(h, wr_ref[...])
    r_ref[...] = _dot(h, wo_ref[...])


def _c_body(sa_ref, sb_ref, rt_ref, x_ref, o_ref):
    o_ref[...] = sa_ref[...] + sb_ref[...] + rt_ref[...] + x_ref[...]


def _sc_scatter(p_hbm, src_hbm, dst_hbm, w_hbm, z_hbm, out_hbm,
                src_v, dst_v,
                dstc0, dstc1, dstc2, wb0, wb1, wb2, rb0, rb1, rb2,
                acc_sh, gs0, gs1, gs2, ss0, ss1, ss2):
    c = lax.axis_index("c")
    s = lax.axis_index("s")
    wid = s * _NC + c
    e0 = wid * _EPT
    pltpu.sync_copy(src_hbm.at[pl.ds(e0, _EPT)], src_v)
    pltpu.sync_copy(dst_hbm.at[pl.ds(e0, _EPT)], dst_v)
    pltpu.sync_copy(z_hbm, rb0)

    def zbody(m, carry):
        k = s + _NS * m

        @pl.when(k < _NRCH - 1)
        def _full():
            pltpu.sync_copy(rb0, acc_sh.at[pl.ds(k * _CHK, _CHK)])

        @pl.when(k == _NRCH - 1)
        def _tail():
            pltpu.sync_copy(rb0.at[pl.ds(0, _RTAIL)],
                            acc_sh.at[pl.ds((_NRCH - 1) * _CHK, _RTAIL)])

        return carry

    lax.fori_loop(0, _MPT, zbody, 0)
    plsc.subcore_barrier()

    bufs = (rb0, rb1, rb2)
    wbufs = (wb0, wb1, wb2)
    dstcs = (dstc0, dstc1, dstc2)
    gsems = (gs0, gs1, gs2)
    ssems = (ss0, ss1, ss2)

    def _gather(j, b):
        return pltpu.make_async_copy(
            p_hbm.at[src_v.at[pl.ds(j * _CHK, _CHK)]], bufs[b], gsems[b])

    def _wcopy(j, b):
        return pltpu.make_async_copy(
            w_hbm.at[pl.ds(e0 + j * _CHK, _CHK)], wbufs[b], gsems[b])

    def _scat(b):
        return pltpu.make_async_copy(bufs[b], acc_sh.at[dstcs[b]], ssems[b])

    for b in range(2):
        _gather(b, b).start()
        _wcopy(b, b).start()

    def _mult(j, b):
        buf = bufs[b]
        wbuf = wbufs[b]
        dstc = dstcs[b]

        def mgrp(g, icarry):
            base = j * _CHK + g * 16
            wv = wbuf[pl.ds(g * 16, 16)]
            dstc[pl.ds(g * 16, 16)] = dst_v[pl.ds(base, 16)]
            for k in range(16):
                wk = wv[k]
                r = g * 16 + k
                for rr in range(8):
                    sl = pl.ds(rr * 16, 16)
                    buf[r, sl] = buf[r, sl] * wk
            return icarry

        lax.fori_loop(0, _CHK // 16, mgrp, 0)

    def _visit(j, b):
        b2 = (b + 2) % 3
        _gather(j, b).wait()
        _wcopy(j, b).wait()
        _mult(j, b)
        _scat(b).start(add=True)

        @pl.when(j >= 1)
        def _wprev():
            _scat(b2).wait()

        @pl.when(j + 2 < _NCHT)
        def _pref():
            _gather(j + 2, b2).start()
            _wcopy(j + 2, b2).start()

    def body(jj, carry):
        for b in range(3):
            _visit(3 * jj + b, b)
        return carry

    lax.fori_loop(0, _NCHT // 3, body, 0)
    jlast = _NCHT - 1
    _gather(jlast, 0).wait()
    _wcopy(jlast, 0).wait()
    _mult(jlast, 0)
    _scat(0).start(add=True)
    _scat(2).wait()
    _scat(0).wait()
    plsc.subcore_barrier()

    def drain(m, carry):
        k = s + _NS * m

        @pl.when(k < _NRCH - 1)
        def _full():
            r0 = k * _CHK
            pltpu.sync_copy(acc_sh.at[pl.ds(r0, _CHK)], rb0)
            pltpu.sync_copy(rb0, out_hbm.at[c].at[pl.ds(r0, _CHK)])

        @pl.when(k == _NRCH - 1)
        def _tail():
            r0 = (_NRCH - 1) * _CHK
            pltpu.sync_copy(acc_sh.at[pl.ds(r0, _RTAIL)], rb0.at[pl.ds(0, _RTAIL)])
            pltpu.sync_copy(rb0.at[pl.ds(0, _RTAIL)],
                            out_hbm.at[c].at[pl.ds(r0, _RTAIL)])

        return carry

    lax.fori_loop(0, _MPT, drain, 0)


def _make_sc_call():
    return pl.kernel(
        _sc_scatter,
        out_type=jax.ShapeDtypeStruct((_NC, _N, _CH), jnp.float32),
        mesh=plsc.VectorSubcoreMesh(core_axis_name="c", subcore_axis_name="s"),
        scratch_types=[
            pltpu.VMEM((_EPT,), jnp.int32),
            pltpu.VMEM((_EPT,), jnp.int32),
            pltpu.VMEM((_CHK,), jnp.int32),
            pltpu.VMEM((_CHK,), jnp.int32),
            pltpu.VMEM((_CHK,), jnp.int32),
            pltpu.VMEM((_CHK,), jnp.float32),
            pltpu.VMEM((_CHK,), jnp.float32),
            pltpu.VMEM((_CHK,), jnp.float32),
            pltpu.VMEM((_CHK, _CH), jnp.float32),
            pltpu.VMEM((_CHK, _CH), jnp.float32),
            pltpu.VMEM((_CHK, _CH), jnp.float32),
            pltpu.VMEM_SHARED((_N, _CH), jnp.float32),
            pltpu.SemaphoreType.DMA,
            pltpu.SemaphoreType.DMA,
            pltpu.SemaphoreType.DMA,
            pltpu.SemaphoreType.DMA,
            pltpu.SemaphoreType.DMA,
            pltpu.SemaphoreType.DMA,
        ],
    )


def _row_spec(nb_lanes=_CH):
    return pl.BlockSpec((_BLK, nb_lanes), lambda i: (i, 0))


def _const_spec(shape):
    return pl.BlockSpec(shape, lambda i: (0,) * len(shape))


def kernel(x, edge_index, edge_weight, batch, t_vec, field, W_t, b_t, W_f,
           W_rel0, W_root0, W_rel1, W_root1, W_m1, b_m1, W_m2, b_m2):
    batch_f = batch.astype(jnp.float32).reshape(_N, 1)
    pad = _EPAD - edge_index.shape[1]
    src_p = jnp.pad(edge_index[0], (0, pad))
    dst_p = jnp.pad(edge_index[1], (0, pad))
    w_p = jnp.pad(edge_weight.reshape(-1), (0, pad))
    zeros = jnp.zeros((_CHK, _CH), jnp.float32)
    bt2 = b_t.reshape(1, -1)
    bm12 = b_m1.reshape(1, -1)
    bm22 = b_m2.reshape(1, -1)

    ms = pl.pallas_call(
        _stats_body,
        out_shape=jax.ShapeDtypeStruct((_G, _CH), jnp.float32),
    )(x, batch_f)

    f32 = jnp.float32
    p0, r0 = pl.pallas_call(
        _a2_body,
        grid=(_NB,),
        in_specs=[
            _row_spec(), _row_spec(1), _row_spec(1),
            _const_spec((_G, _CH)), _const_spec((_G, _CH)),
            _const_spec((_CH, 2 * _CH)), _const_spec((1, 2 * _CH)),
            _const_spec((1, 2 * _CH)),
            _const_spec((_CH, 2 * _CH)), _const_spec((1, 2 * _CH)),
            _const_spec((2 * _CH, _CH)), _const_spec((1, _CH)),
            _const_spec((_CH, _CH)), _const_spec((_CH, _CH)),
        ],
        out_specs=[_row_spec(), _row_spec()],
        out_shape=[jax.ShapeDtypeStruct((_N, _CH), f32),
                   jax.ShapeDtypeStruct((_N, _CH), f32)],
    )(x, batch_f, field, ms, t_vec, W_t, bt2, W_f, W_m1, bm12, W_m2, bm22,
      W_rel0, W_root0)

    sc_call = _make_sc_call()
    s0 = sc_call(p0, src_p, dst_p, w_p, zeros)

    p1, r1 = pl.pallas_call(
        _b_body,
        grid=(_NB,),
        in_specs=[
            _row_spec(), _row_spec(), _row_spec(),
            _const_spec((_CH, _CH)), _const_spec((_CH, _CH)),
        ],
        out_specs=[_row_spec(), _row_spec()],
        out_shape=[jax.ShapeDtypeStruct((_N, _CH), f32),
                   jax.ShapeDtypeStruct((_N, _CH), f32)],
    )(s0[0], s0[1], r0, W_rel1, W_root1)

    s1 = sc_call(p1, src_p, dst_p, w_p, zeros)

    out = pl.pallas_call(
        _c_body,
        grid=(_NB,),
        in_specs=[_row_spec(), _row_spec(), _row_spec(), _row_spec()],
        out_specs=_row_spec(),
        out_shape=jax.ShapeDtypeStruct((_N, _CH), f32),
    )(s1[0], s1[1], r1, x)
    return out


# speedup vs baseline: 1.2565x; 1.2565x over previous
import jax
import jax.numpy as jnp
from jax import lax
from jax.experimental import pallas as pl
from jax.experimental.pallas import tpu as pltpu
from jax.experimental.pallas import tpu_sc as plsc

_N = 10000
_CH = 128
_G = 16
_EPS = 1e-6

_NC = 2
_NS = 16
_NW = _NC * _NS
_CHK = 80
_NCHT = 125
_EPT = _NCHT * _CHK
_EPAD = _NW * _EPT
_NRCH = _N // _CHK
_RTAIL = _CHK
_MPT = 8

_BLK = 2000
_NB = _N // _BLK

_HI = lax.Precision.HIGHEST


def _sig(z):
    return 1.0 / (1.0 + jnp.exp(-z))


def _dot(a, b):
    return lax.dot_general(a, b, (((1,), (0,)), ((), ())), precision=_HI)


def _dotT(a, b):
    return lax.dot_general(a, b, (((0,), (0,)), ((), ())), precision=_HI)


def _onehot(b):
    ids = lax.broadcasted_iota(jnp.int32, (1, _G), 1).astype(jnp.float32)
    return (b == ids).astype(jnp.float32)


def _stats_body(x_ref, b_ref, ms_ref):
    x = x_ref[...]
    oh = _onehot(b_ref[...])
    stats = _dotT(oh, x * x)
    counts = _dotT(oh, jnp.ones_like(b_ref[...]))
    ms_ref[...] = stats / jnp.maximum(counts, 1.0)


def _a2_body(x_ref, b_ref, f_ref, ms_ref, t_ref, wt_ref, bt_ref, wf_ref,
             wm1_ref, bm1_ref, wm2_ref, bm2_ref, wr_ref, wo_ref,
             p_ref, r_ref):
    x = x_ref[...]
    oh = _onehot(b_ref[...])
    inv = lax.rsqrt(_dot(oh, ms_ref[...]) + _EPS)
    h = x * inv
    tv = t_ref[...]
    st = _dot(tv * _sig(tv), wt_ref[...]) + bt_ref[...]
    cond = _dot(oh, st) + f_ref[...] * wf_ref[...]
    gamma = cond[:, :_CH]
    beta = cond[:, _CH:]
    h = h * (1.0 + gamma) + beta
    h = h * _sig(h)
    u = _dot(h, wm1_ref[...]) + bm1_ref[...]
    u = u * _sig(u)
    h2 = h + _dot(u, wm2_ref[...]) + bm2_ref[...]
    p_ref[...] = _dot(h2, wr_ref[...])
    r_ref[...] = _dot(h2, wo_ref[...])


def _b_body(sa_ref, sb_ref, rt_ref, wr_ref, wo_ref, p_ref, r_ref):
    h = sa_ref[...] + sb_ref[...] + rt_ref[...]
    h = h * _sig(h)
    p_ref[...] = _dot(h, wr_ref[...])
    r_ref[...] = _dot(h, wo_ref[...])


def _c_body(sa_ref, sb_ref, rt_ref, x_ref, o_ref):
    o_ref[...] = sa_ref[...] + sb_ref[...] + rt_ref[...] + x_ref[...]


def _sc_scatter(p_hbm, src_hbm, dst_hbm, w_hbm, z_hbm, out_hbm,
                src_v, dst_v,
                dstc0, dstc1, wb0, wb1, rb0, rb1,
                acc_sh, gs0, gs1):
    c = lax.axis_index("c")
    s = lax.axis_index("s")
    wid = s * _NC + c
    e0 = wid * _EPT
    pltpu.sync_copy(src_hbm.at[pl.ds(e0, _EPT)], src_v)
    pltpu.sync_copy(dst_hbm.at[pl.ds(e0, _EPT)], dst_v)
    pltpu.sync_copy(z_hbm, rb0)

    def zbody(m, carry):
        k = s + _NS * m

        @pl.when(k < _NRCH - 1)
        def _full():
            pltpu.sync_copy(rb0, acc_sh.at[pl.ds(k * _CHK, _CHK)])

        @pl.when(k == _NRCH - 1)
        def _tail():
            pltpu.sync_copy(rb0.at[pl.ds(0, _RTAIL)],
                            acc_sh.at[pl.ds((_NRCH - 1) * _CHK, _RTAIL)])

        return carry

    lax.fori_loop(0, _MPT, zbody, 0)
    plsc.subcore_barrier()

    bufs = (rb0, rb1)
    wbufs = (wb0, wb1)
    dstcs = (dstc0, dstc1)
    gsems = (gs0, gs1)

    def _gather(j, b):
        return pltpu.make_async_copy(
            p_hbm.at[src_v.at[pl.ds(j * _CHK, _CHK)]], bufs[b], gsems[b])

    def _wcopy(j, b):
        return pltpu.make_async_copy(
            w_hbm.at[pl.ds(e0 + j * _CHK, _CHK)], wbufs[b], gsems[b])

    for b in range(2):
        _gather(b, b).start()
        _wcopy(b, b).start()

    def _mult(j, b):
        buf = bufs[b]
        wbuf = wbufs[b]
        dstc = dstcs[b]

        def mgrp(g, icarry):
            base = j * _CHK + g * 16
            wv = wbuf[pl.ds(g * 16, 16)]
            dstc[pl.ds(g * 16, 16)] = dst_v[pl.ds(base, 16)]
            for k in range(16):
                wk = wv[k]
                r = g * 16 + k
                for rr in range(8):
                    sl = pl.ds(rr * 16, 16)
                    buf[r, sl] = buf[r, sl] * wk
            return icarry

        lax.fori_loop(0, _CHK // 16, mgrp, 0)

    def _visit(j, b):
        _gather(j, b).wait()
        _wcopy(j, b).wait()
        _mult(j, b)
        pltpu.sync_copy(bufs[b], acc_sh.at[dstcs[b]], add=True)

        @pl.when(j + 2 < _NCHT)
        def _pref():
            _gather(j + 2, b).start()
            _wcopy(j + 2, b).start()

    def body(jj, carry):
        for b in range(2):
            _visit(2 * jj + b, b)
        return carry

    lax.fori_loop(0, _NCHT // 2, body, 0)
    jlast = _NCHT - 1
    _gather(jlast, 0).wait()
    _wcopy(jlast, 0).wait()
    _mult(jlast, 0)
    pltpu.sync_copy(bufs[0], acc_sh.at[dstcs[0]], add=True)
    plsc.subcore_barrier()

    def drain(m, carry):
        k = s + _NS * m

        @pl.when(k < _NRCH - 1)
        def _full():
            r0 = k * _CHK
            pltpu.sync_copy(acc_sh.at[pl.ds(r0, _CHK)], rb0)
            pltpu.sync_copy(rb0, out_hbm.at[c].at[pl.ds(r0, _CHK)])

        @pl.when(k == _NRCH - 1)
        def _tail():
            r0 = (_NRCH - 1) * _CHK
            pltpu.sync_copy(acc_sh.at[pl.ds(r0, _RTAIL)], rb0.at[pl.ds(0, _RTAIL)])
            pltpu.sync_copy(rb0.at[pl.ds(0, _RTAIL)],
                            out_hbm.at[c].at[pl.ds(r0, _RTAIL)])

        return carry

    lax.fori_loop(0, _MPT, drain, 0)


def _make_sc_call():
    return pl.kernel(
        _sc_scatter,
        out_type=jax.ShapeDtypeStruct((_NC, _N, _CH), jnp.float32),
        mesh=plsc.VectorSubcoreMesh(core_axis_name="c", subcore_axis_name="s"),
        scratch_types=[
            pltpu.VMEM((_EPT,), jnp.int32),
            pltpu.VMEM((_EPT,), jnp.int32),
            pltpu.VMEM((_CHK,), jnp.int32),
            pltpu.VMEM((_CHK,), jnp.int32),
            pltpu.VMEM((_CHK,), jnp.float32),
            pltpu.VMEM((_CHK,), jnp.float32),
            pltpu.VMEM((_CHK, _CH), jnp.float32),
            pltpu.VMEM((_CHK, _CH), jnp.float32),
            pltpu.VMEM_SHARED((_N, _CH), jnp.float32),
            pltpu.SemaphoreType.DMA,
            pltpu.SemaphoreType.DMA,
        ],
    )


def _row_spec(nb_lanes=_CH):
    return pl.BlockSpec((_BLK, nb_lanes), lambda i: (i, 0))


def _const_spec(shape):
    return pl.BlockSpec(shape, lambda i: (0,) * len(shape))


def kernel(x, edge_index, edge_weight, batch, t_vec, field, W_t, b_t, W_f,
           W_rel0, W_root0, W_rel1, W_root1, W_m1, b_m1, W_m2, b_m2):
    batch_f = batch.astype(jnp.float32).reshape(_N, 1)
    pad = _EPAD - edge_index.shape[1]
    src_p = jnp.pad(edge_index[0], (0, pad))
    dst_p = jnp.pad(edge_index[1], (0, pad))
    w_p = jnp.pad(edge_weight.reshape(-1), (0, pad))
    zeros = jnp.zeros((_CHK, _CH), jnp.float32)
    bt2 = b_t.reshape(1, -1)
    bm12 = b_m1.reshape(1, -1)
    bm22 = b_m2.reshape(1, -1)

    ms = pl.pallas_call(
        _stats_body,
        out_shape=jax.ShapeDtypeStruct((_G, _CH), jnp.float32),
    )(x, batch_f)

    f32 = jnp.float32
    p0, r0 = pl.pallas_call(
        _a2_body,
        grid=(_NB,),
        in_specs=[
            _row_spec(), _row_spec(1), _row_spec(1),
            _const_spec((_G, _CH)), _const_spec((_G, _CH)),
            _const_spec((_CH, 2 * _CH)), _const_spec((1, 2 * _CH)),
            _const_spec((1, 2 * _CH)),
            _const_spec((_CH, 2 * _CH)), _const_spec((1, 2 * _CH)),
            _const_spec((2 * _CH, _CH)), _const_spec((1, _CH)),
            _const_spec((_CH, _CH)), _const_spec((_CH, _CH)),
        ],
        out_specs=[_row_spec(), _row_spec()],
        out_shape=[jax.ShapeDtypeStruct((_N, _CH), f32),
                   jax.ShapeDtypeStruct((_N, _CH), f32)],
    )(x, batch_f, field, ms, t_vec, W_t, bt2, W_f, W_m1, bm12, W_m2, bm22,
      W_rel0, W_root0)

    sc_call = _make_sc_call()
    s0 = sc_call(p0, src_p, dst_p, w_p, zeros)

    p1, r1 = pl.pallas_call(
        _b_body,
        grid=(_NB,),
        in_specs=[
            _row_spec(), _row_spec(), _row_spec(),
            _const_spec((_CH, _CH)), _const_spec((_CH, _CH)),
        ],
        out_specs=[_row_spec(), _row_spec()],
        out_shape=[jax.ShapeDtypeStruct((_N, _CH), f32),
                   jax.ShapeDtypeStruct((_N, _CH), f32)],
    )(s0[0], s0[1], r0, W_rel1, W_root1)

    s1 = sc_call(p1, src_p, dst_p, w_p, zeros)

    out = pl.pallas_call(
        _c_body,
        grid=(_NB,),
        in_specs=[_row_spec(), _row_spec(), _row_spec(), _row_spec()],
        out_specs=_row_spec(),
        out_shape=jax.ShapeDtypeStruct((_N, _CH), f32),
    )(s1[0], s1[1], r1, x)
    return out


# speedup vs baseline: 1.3759x; 1.0950x over previous
import jax
import jax.numpy as jnp
from jax import lax
from jax.experimental import pallas as pl
from jax.experimental.pallas import tpu as pltpu
from jax.experimental.pallas import tpu_sc as plsc

_N = 10000
_CH = 128
_G = 16
_EPS = 1e-6

_NC = 2
_NS = 16
_NW = _NC * _NS
_CHK = 80
_NCHT = 125
_EPT = _NCHT * _CHK
_EPAD = _NW * _EPT
_NRCH = _N // _CHK
_RTAIL = _CHK
_MPT = 8

_BLK = 2000
_NB = _N // _BLK

_HI = lax.Precision.HIGHEST


def _sig(z):
    return 1.0 / (1.0 + jnp.exp(-z))


def _dot(a, b):
    return lax.dot_general(a, b, (((1,), (0,)), ((), ())), precision=_HI)


def _dotT(a, b):
    return lax.dot_general(a, b, (((0,), (0,)), ((), ())), precision=_HI)


def _onehot(b):
    ids = lax.broadcasted_iota(jnp.int32, (1, _G), 1).astype(jnp.float32)
    return (b == ids).astype(jnp.float32)


def _stats_body(x_ref, b_ref, ms_ref):
    x = x_ref[...]
    oh = _onehot(b_ref[...])
    stats = _dotT(oh, x * x)
    counts = _dotT(oh, jnp.ones_like(b_ref[...]))
    ms_ref[...] = stats / jnp.maximum(counts, 1.0)


def _a2_body(x_ref, b_ref, f_ref, ms_ref, t_ref, wt_ref, bt_ref, wf_ref,
             wm1_ref, bm1_ref, wm2_ref, bm2_ref, wr_ref, wo_ref,
             p_ref, r_ref):
    x = x_ref[...]
    oh = _onehot(b_ref[...])
    inv = lax.rsqrt(_dot(oh, ms_ref[...]) + _EPS)
    h = x * inv
    tv = t_ref[...]
    st = _dot(tv * _sig(tv), wt_ref[...]) + bt_ref[...]
    cond = _dot(oh, st) + f_ref[...] * wf_ref[...]
    gamma = cond[:, :_CH]
    beta = cond[:, _CH:]
    h = h * (1.0 + gamma) + beta
    h = h * _sig(h)
    u = _dot(h, wm1_ref[...]) + bm1_ref[...]
    u = u * _sig(u)
    h2 = h + _dot(u, wm2_ref[...]) + bm2_ref[...]
    p_ref[...] = _dot(h2, wr_ref[...])
    r_ref[...] = _dot(h2, wo_ref[...])


def _b_body(sa_ref, sb_ref, rt_ref, wr_ref, wo_ref, p_ref, r_ref):
    h = sa_ref[...] + sb_ref[...] + rt_ref[...]
    h = h * _sig(h)
    p_ref[...] = _dot(h, wr_ref[...])
    r_ref[...] = _dot(h, wo_ref[...])


def _c_body(sa_ref, sb_ref, rt_ref, x_ref, o_ref):
    o_ref[...] = sa_ref[...] + sb_ref[...] + rt_ref[...] + x_ref[...]


def _sc_scatter(p_hbm, src_hbm, dst_hbm, w_hbm, z_hbm, out_hbm,
                src_v,
                dstc0, dstc1, dstc2, wb0, wb1, wb2, rb0, rb1, rb2,
                acc_sh, gs0, gs1, gs2, ss0, ss1, ss2):
    c = lax.axis_index("c")
    s = lax.axis_index("s")
    wid = s * _NC + c
    e0 = wid * _EPT
    pltpu.sync_copy(src_hbm.at[pl.ds(e0, _EPT)], src_v)
    pltpu.sync_copy(z_hbm, rb0)

    def zbody(m, carry):
        k = s + _NS * m

        @pl.when(k < _NRCH - 1)
        def _full():
            pltpu.sync_copy(rb0, acc_sh.at[pl.ds(k * _CHK, _CHK)])

        @pl.when(k == _NRCH - 1)
        def _tail():
            pltpu.sync_copy(rb0.at[pl.ds(0, _RTAIL)],
                            acc_sh.at[pl.ds((_NRCH - 1) * _CHK, _RTAIL)])

        return carry

    lax.fori_loop(0, _MPT, zbody, 0)
    plsc.subcore_barrier()

    bufs = (rb0, rb1, rb2)
    wbufs = (wb0, wb1, wb2)
    dstcs = (dstc0, dstc1, dstc2)
    gsems = (gs0, gs1, gs2)
    ssems = (ss0, ss1, ss2)

    def _gather(j, b):
        return pltpu.make_async_copy(
            p_hbm.at[src_v.at[pl.ds(j * _CHK, _CHK)]], bufs[b], gsems[b])

    def _wcopy(j, b):
        return pltpu.make_async_copy(
            w_hbm.at[pl.ds(e0 + j * _CHK, _CHK)], wbufs[b], gsems[b])

    def _dcopy(j, b):
        return pltpu.make_async_copy(
            dst_hbm.at[pl.ds(e0 + j * _CHK, _CHK)], dstcs[b], gsems[b])

    def _scat(b):
        return pltpu.make_async_copy(bufs[b], acc_sh.at[dstcs[b]], ssems[b])

    for b in range(2):
        _gather(b, b).start()
        _wcopy(b, b).start()
        _dcopy(b, b).start()

    def _mult(j, b):
        buf = bufs[b]
        wbuf = wbufs[b]

        def mgrp(g, icarry):
            wv = wbuf[pl.ds(g * 16, 16)]
            for k in range(16):
                wk = wv[k]
                r = g * 16 + k
                for rr in range(8):
                    sl = pl.ds(rr * 16, 16)
                    buf[r, sl] = buf[r, sl] * wk
            return icarry

        lax.fori_loop(0, _CHK // 16, mgrp, 0)

    def _visit(j, b):
        b2 = (b + 2) % 3
        _gather(j, b).wait()
        _wcopy(j, b).wait()
        _dcopy(j, b).wait()
        _mult(j, b)
        _scat(b).start(add=True)

        @pl.when(j >= 1)
        def _wprev():
            _scat(b2).wait()

        @pl.when(j + 2 < _NCHT)
        def _pref():
            _gather(j + 2, b2).start()
            _wcopy(j + 2, b2).start()
            _dcopy(j + 2, b2).start()

    def body(jj, carry):
        for b in range(3):
            _visit(3 * jj + b, b)
        return carry

    nloop = _NCHT // 3
    lax.fori_loop(0, nloop, body, 0)
    for j, b in ((_NCHT - 2, 0), (_NCHT - 1, 1)):
        _gather(j, b).wait()
        _wcopy(j, b).wait()
        _dcopy(j, b).wait()
        _mult(j, b)
        _scat(b).start(add=True)
        _scat((b + 2) % 3).wait()
    _scat(1).wait()
    plsc.subcore_barrier()

    def drain(m, carry):
        k = s + _NS * m

        @pl.when(k < _NRCH - 1)
        def _full():
            r0 = k * _CHK
            pltpu.sync_copy(acc_sh.at[pl.ds(r0, _CHK)], rb0)
            pltpu.sync_copy(rb0, out_hbm.at[c].at[pl.ds(r0, _CHK)])

        @pl.when(k == _NRCH - 1)
        def _tail():
            r0 = (_NRCH - 1) * _CHK
            pltpu.sync_copy(acc_sh.at[pl.ds(r0, _RTAIL)], rb0.at[pl.ds(0, _RTAIL)])
            pltpu.sync_copy(rb0.at[pl.ds(0, _RTAIL)],
                            out_hbm.at[c].at[pl.ds(r0, _RTAIL)])

        return carry

    lax.fori_loop(0, _MPT, drain, 0)


def _make_sc_call():
    return pl.kernel(
        _sc_scatter,
        out_type=jax.ShapeDtypeStruct((_NC, _N, _CH), jnp.float32),
        mesh=plsc.VectorSubcoreMesh(core_axis_name="c", subcore_axis_name="s"),
        scratch_types=[
            pltpu.VMEM((_EPT,), jnp.int32),
            pltpu.VMEM((_CHK,), jnp.int32),
            pltpu.VMEM((_CHK,), jnp.int32),
            pltpu.VMEM((_CHK,), jnp.int32),
            pltpu.VMEM((_CHK,), jnp.float32),
            pltpu.VMEM((_CHK,), jnp.float32),
            pltpu.VMEM((_CHK,), jnp.float32),
            pltpu.VMEM((_CHK, _CH), jnp.float32),
            pltpu.VMEM((_CHK, _CH), jnp.float32),
            pltpu.VMEM((_CHK, _CH), jnp.float32),
            pltpu.VMEM_SHARED((_N, _CH), jnp.float32),
            pltpu.SemaphoreType.DMA,
            pltpu.SemaphoreType.DMA,
            pltpu.SemaphoreType.DMA,
            pltpu.SemaphoreType.DMA,
            pltpu.SemaphoreType.DMA,
            pltpu.SemaphoreType.DMA,
        ],
    )


def _row_spec(nb_lanes=_CH):
    return pl.BlockSpec((_BLK, nb_lanes), lambda i: (i, 0))


def _const_spec(shape):
    return pl.BlockSpec(shape, lambda i: (0,) * len(shape))


def kernel(x, edge_index, edge_weight, batch, t_vec, field, W_t, b_t, W_f,
           W_rel0, W_root0, W_rel1, W_root1, W_m1, b_m1, W_m2, b_m2):
    batch_f = batch.astype(jnp.float32).reshape(_N, 1)
    pad = _EPAD - edge_index.shape[1]
    src_p = jnp.pad(edge_index[0], (0, pad))
    dst_p = jnp.pad(edge_index[1], (0, pad))
    w_p = jnp.pad(edge_weight.reshape(-1), (0, pad))
    zeros = jnp.zeros((_CHK, _CH), jnp.float32)
    bt2 = b_t.reshape(1, -1)
    bm12 = b_m1.reshape(1, -1)
    bm22 = b_m2.reshape(1, -1)

    ms = pl.pallas_call(
        _stats_body,
        out_shape=jax.ShapeDtypeStruct((_G, _CH), jnp.float32),
    )(x, batch_f)

    f32 = jnp.float32
    p0, r0 = pl.pallas_call(
        _a2_body,
        grid=(_NB,),
        in_specs=[
            _row_spec(), _row_spec(1), _row_spec(1),
            _const_spec((_G, _CH)), _const_spec((_G, _CH)),
            _const_spec((_CH, 2 * _CH)), _const_spec((1, 2 * _CH)),
            _const_spec((1, 2 * _CH)),
            _const_spec((_CH, 2 * _CH)), _const_spec((1, 2 * _CH)),
            _const_spec((2 * _CH, _CH)), _const_spec((1, _CH)),
            _const_spec((_CH, _CH)), _const_spec((_CH, _CH)),
        ],
        out_specs=[_row_spec(), _row_spec()],
        out_shape=[jax.ShapeDtypeStruct((_N, _CH), f32),
                   jax.ShapeDtypeStruct((_N, _CH), f32)],
    )(x, batch_f, field, ms, t_vec, W_t, bt2, W_f, W_m1, bm12, W_m2, bm22,
      W_rel0, W_root0)

    sc_call = _make_sc_call()
    s0 = sc_call(p0, src_p, dst_p, w_p, zeros)

    p1, r1 = pl.pallas_call(
        _b_body,
        grid=(_NB,),
        in_specs=[
            _row_spec(), _row_spec(), _row_spec(),
            _const_spec((_CH, _CH)), _const_spec((_CH, _CH)),
        ],
        out_specs=[_row_spec(), _row_spec()],
        out_shape=[jax.ShapeDtypeStruct((_N, _CH), f32),
                   jax.ShapeDtypeStruct((_N, _CH), f32)],
    )(s0[0], s0[1], r0, W_rel1, W_root1)

    s1 = sc_call(p1, src_p, dst_p, w_p, zeros)

    out = pl.pallas_call(
        _c_body,
        grid=(_NB,),
        in_specs=[_row_spec(), _row_spec(), _row_spec(), _row_spec()],
        out_specs=_row_spec(),
        out_shape=jax.ShapeDtypeStruct((_N, _CH), f32),
    )(s1[0], s1[1], r1, x)
    return out


# speedup vs baseline: 1.6795x; 1.2206x over previous
import jax
import jax.numpy as jnp
from jax import lax
from jax.experimental import pallas as pl
from jax.experimental.pallas import tpu as pltpu
from jax.experimental.pallas import tpu_sc as plsc

_N = 10000
_CH = 128
_G = 16
_EPS = 1e-6

_NC = 2
_NS = 16
_NW = _NC * _NS
_CHK = 80
_NCHT = 125
_EPT = _NCHT * _CHK
_EPAD = _NW * _EPT
_NRCH = _N // _CHK
_RTAIL = _CHK
_MPT = 8

_BLK = 2000
_NB = _N // _BLK

_HI = lax.Precision.DEFAULT


def _sig(z):
    return 1.0 / (1.0 + jnp.exp(-z))


def _dot(a, b):
    return lax.dot_general(a, b, (((1,), (0,)), ((), ())), precision=_HI)


def _dotT(a, b):
    return lax.dot_general(a, b, (((0,), (0,)), ((), ())), precision=_HI)


def _onehot(b):
    ids = lax.broadcasted_iota(jnp.int32, (1, _G), 1).astype(jnp.float32)
    return (b == ids).astype(jnp.float32)


def _stats_body(x_ref, b_ref, ms_ref):
    x = x_ref[...]
    oh = _onehot(b_ref[...])
    stats = _dotT(oh, x * x)
    counts = _dotT(oh, jnp.ones_like(b_ref[...]))
    ms_ref[...] = stats / jnp.maximum(counts, 1.0)


def _a2_body(x_ref, b_ref, f_ref, ms_ref, t_ref, wt_ref, bt_ref, wf_ref,
             wm1_ref, bm1_ref, wm2_ref, bm2_ref, wr_ref, wo_ref,
             p_ref, r_ref):
    x = x_ref[...]
    oh = _onehot(b_ref[...])
    inv = lax.rsqrt(_dot(oh, ms_ref[...]) + _EPS)
    h = x * inv
    tv = t_ref[...]
    st = _dot(tv * _sig(tv), wt_ref[...]) + bt_ref[...]
    cond = _dot(oh, st) + f_ref[...] * wf_ref[...]
    gamma = cond[:, :_CH]
    beta = cond[:, _CH:]
    h = h * (1.0 + gamma) + beta
    h = h * _sig(h)
    u = _dot(h, wm1_ref[...]) + bm1_ref[...]
    u = u * _sig(u)
    h2 = h + _dot(u, wm2_ref[...]) + bm2_ref[...]
    p_ref[...] = _dot(h2, wr_ref[...])
    r_ref[...] = _dot(h2, wo_ref[...])


def _b_body(sa_ref, sb_ref, rt_ref, wr_ref, wo_ref, p_ref, r_ref):
    h = sa_ref[0] + sb_ref[0] + rt_ref[...]
    h = h * _sig(h)
    p_ref[...] = _dot(h, wr_ref[...])
    r_ref[...] = _dot(h, wo_ref[...])


def _c_body(sa_ref, sb_ref, rt_ref, x_ref, o_ref):
    o_ref[...] = sa_ref[0] + sb_ref[0] + rt_ref[...] + x_ref[...]


def _sc_scatter(p_hbm, ei_hbm, w_hbm, z_hbm, out_hbm,
                src_v,
                dstc0, dstc1, dstc2, wb0, wb1, wb2, rb0, rb1, rb2,
                acc_sh, gs0, gs1, gs2, ss0, ss1, ss2):
    c = lax.axis_index("c")
    s = lax.axis_index("s")
    wid = s * _NC + c
    e0 = wid * _EPT
    pltpu.sync_copy(ei_hbm.at[pl.ds(e0, _EPT)], src_v)
    pltpu.sync_copy(z_hbm, rb0)

    def zbody(m, carry):
        k = s + _NS * m

        @pl.when(k < _NRCH - 1)
        def _full():
            pltpu.sync_copy(rb0, acc_sh.at[pl.ds(k * _CHK, _CHK)])

        @pl.when(k == _NRCH - 1)
        def _tail():
            pltpu.sync_copy(rb0.at[pl.ds(0, _RTAIL)],
                            acc_sh.at[pl.ds((_NRCH - 1) * _CHK, _RTAIL)])

        return carry

    lax.fori_loop(0, _MPT, zbody, 0)
    plsc.subcore_barrier()

    bufs = (rb0, rb1, rb2)
    wbufs = (wb0, wb1, wb2)
    dstcs = (dstc0, dstc1, dstc2)
    gsems = (gs0, gs1, gs2)
    ssems = (ss0, ss1, ss2)

    def _gather(j, b):
        return pltpu.make_async_copy(
            p_hbm.at[src_v.at[pl.ds(j * _CHK, _CHK)]], bufs[b], gsems[b])

    def _wcopy(j, b):
        return pltpu.make_async_copy(
            w_hbm.at[pl.ds(e0 + j * _CHK, _CHK)], wbufs[b], gsems[b])

    def _dcopy(j, b):
        return pltpu.make_async_copy(
            ei_hbm.at[pl.ds(_EPAD + e0 + j * _CHK, _CHK)], dstcs[b], gsems[b])

    def _scat(b):
        return pltpu.make_async_copy(bufs[b], acc_sh.at[dstcs[b]], ssems[b])

    for b in range(2):
        _gather(b, b).start()
        _wcopy(b, b).start()
        _dcopy(b, b).start()

    def _mult(j, b):
        buf = bufs[b]
        wbuf = wbufs[b]

        def mgrp(g, icarry):
            wv = wbuf[pl.ds(g * 16, 16)]
            for k in range(16):
                wk = wv[k]
                r = g * 16 + k
                for rr in range(8):
                    sl = pl.ds(rr * 16, 16)
                    buf[r, sl] = buf[r, sl] * wk
            return icarry

        lax.fori_loop(0, _CHK // 16, mgrp, 0)

    def _visit(j, b):
        b2 = (b + 2) % 3
        _gather(j, b).wait()
        _wcopy(j, b).wait()
        _dcopy(j, b).wait()
        _mult(j, b)
        _scat(b).start(add=True)

        @pl.when(j >= 1)
        def _wprev():
            _scat(b2).wait()

        @pl.when(j + 2 < _NCHT)
        def _pref():
            _gather(j + 2, b2).start()
            _wcopy(j + 2, b2).start()
            _dcopy(j + 2, b2).start()

    def body(jj, carry):
        for b in range(3):
            _visit(3 * jj + b, b)
        return carry

    nloop = _NCHT // 3
    lax.fori_loop(0, nloop, body, 0)
    for j, b in ((_NCHT - 2, 0), (_NCHT - 1, 1)):
        _gather(j, b).wait()
        _wcopy(j, b).wait()
        _dcopy(j, b).wait()
        _mult(j, b)
        _scat(b).start(add=True)
        _scat((b + 2) % 3).wait()
    _scat(1).wait()
    plsc.subcore_barrier()

    def drain(m, carry):
        k = s + _NS * m

        @pl.when(k < _NRCH - 1)
        def _full():
            r0 = k * _CHK
            pltpu.sync_copy(acc_sh.at[pl.ds(r0, _CHK)], rb0)
            pltpu.sync_copy(rb0, out_hbm.at[c].at[pl.ds(r0, _CHK)])

        @pl.when(k == _NRCH - 1)
        def _tail():
            r0 = (_NRCH - 1) * _CHK
            pltpu.sync_copy(acc_sh.at[pl.ds(r0, _RTAIL)], rb0.at[pl.ds(0, _RTAIL)])
            pltpu.sync_copy(rb0.at[pl.ds(0, _RTAIL)],
                            out_hbm.at[c].at[pl.ds(r0, _RTAIL)])

        return carry

    lax.fori_loop(0, _MPT, drain, 0)


def _make_sc_call():
    return pl.kernel(
        _sc_scatter,
        out_type=jax.ShapeDtypeStruct((_NC, _N, _CH), jnp.float32),
        mesh=plsc.VectorSubcoreMesh(core_axis_name="c", subcore_axis_name="s"),
        scratch_types=[
            pltpu.VMEM((_EPT,), jnp.int32),
            pltpu.VMEM((_CHK,), jnp.int32),
            pltpu.VMEM((_CHK,), jnp.int32),
            pltpu.VMEM((_CHK,), jnp.int32),
            pltpu.VMEM((_CHK,), jnp.float32),
            pltpu.VMEM((_CHK,), jnp.float32),
            pltpu.VMEM((_CHK,), jnp.float32),
            pltpu.VMEM((_CHK, _CH), jnp.float32),
            pltpu.VMEM((_CHK, _CH), jnp.float32),
            pltpu.VMEM((_CHK, _CH), jnp.float32),
            pltpu.VMEM_SHARED((_N, _CH), jnp.float32),
            pltpu.SemaphoreType.DMA,
            pltpu.SemaphoreType.DMA,
            pltpu.SemaphoreType.DMA,
            pltpu.SemaphoreType.DMA,
            pltpu.SemaphoreType.DMA,
            pltpu.SemaphoreType.DMA,
        ],
    )


def _row_spec(nb_lanes=_CH):
    return pl.BlockSpec((_BLK, nb_lanes), lambda i: (i, 0))


def _const_spec(shape):
    return pl.BlockSpec(shape, lambda i: (0,) * len(shape))


def kernel(x, edge_index, edge_weight, batch, t_vec, field, W_t, b_t, W_f,
           W_rel0, W_root0, W_rel1, W_root1, W_m1, b_m1, W_m2, b_m2):
    batch_f = batch.astype(jnp.float32).reshape(_N, 1)
    ei_flat = edge_index.reshape(-1)
    w_p = edge_weight.reshape(-1)
    zeros = jnp.zeros((_CHK, _CH), jnp.float32)
    bt2 = b_t.reshape(1, -1)
    bm12 = b_m1.reshape(1, -1)
    bm22 = b_m2.reshape(1, -1)

    ms = pl.pallas_call(
        _stats_body,
        out_shape=jax.ShapeDtypeStruct((_G, _CH), jnp.float32),
    )(x, batch_f)

    f32 = jnp.float32
    p0, r0 = pl.pallas_call(
        _a2_body,
        grid=(_NB,),
        in_specs=[
            _row_spec(), _row_spec(1), _row_spec(1),
            _const_spec((_G, _CH)), _const_spec((_G, _CH)),
            _const_spec((_CH, 2 * _CH)), _const_spec((1, 2 * _CH)),
            _const_spec((1, 2 * _CH)),
            _const_spec((_CH, 2 * _CH)), _const_spec((1, 2 * _CH)),
            _const_spec((2 * _CH, _CH)), _const_spec((1, _CH)),
            _const_spec((_CH, _CH)), _const_spec((_CH, _CH)),
        ],
        out_specs=[_row_spec(), _row_spec()],
        out_shape=[jax.ShapeDtypeStruct((_N, _CH), f32),
                   jax.ShapeDtypeStruct((_N, _CH), f32)],
    )(x, batch_f, field, ms, t_vec, W_t, bt2, W_f, W_m1, bm12, W_m2, bm22,
      W_rel0, W_root0)

    part_a = pl.BlockSpec((1, _BLK, _CH), lambda i: (0, i, 0))
    part_b = pl.BlockSpec((1, _BLK, _CH), lambda i: (1, i, 0))

    sc_call = _make_sc_call()
    s0 = sc_call(p0, ei_flat, w_p, zeros)

    p1, r1 = pl.pallas_call(
        _b_body,
        grid=(_NB,),
        in_specs=[
            part_a, part_b, _row_spec(),
            _const_spec((_CH, _CH)), _const_spec((_CH, _CH)),
        ],
        out_specs=[_row_spec(), _row_spec()],
        out_shape=[jax.ShapeDtypeStruct((_N, _CH), f32),
                   jax.ShapeDtypeStruct((_N, _CH), f32)],
    )(s0, s0, r0, W_rel1, W_root1)

    s1 = sc_call(p1, ei_flat, w_p, zeros)

    out = pl.pallas_call(
        _c_body,
        grid=(_NB,),
        in_specs=[part_a, part_b, _row_spec(), _row_spec()],
        out_specs=_row_spec(),
        out_shape=jax.ShapeDtypeStruct((_N, _CH), f32),
    )(s1, s1, r1, x)
    return out
